# baseline (speedup 1.0000x reference)
import functools

import jax
import jax.numpy as jnp
from jax import lax
from jax.experimental import pallas as pl
from jax.experimental.pallas import tpu as pltpu
from jax.experimental.pallas import tpu_sc as plsc

KCLS = 128
BATCH = 64
PIX = 32 * 32
OUT_WORDS = BATCH * KCLS * PIX

NC = 2
NS = 16
NW = NC * NS
BPW = BATCH // NW
WREG = BPW * KCLS * PIX
XW = BPW * PIX
ZWORDS = 32768
NZD = WREG // ZWORDS
NSCAT = XW // 128


def _sc_body(x_hbm, out_hbm, xv, idxv, zbuf, ones, semz, sems):
    cid = lax.axis_index("c")
    sid = lax.axis_index("s")
    wid = sid * NC + cid
    base = wid * WREG

    @pl.loop(0, ZWORDS // 16)
    def _zfill(i):
        zbuf[pl.ds(i * 16, 16)] = jnp.zeros((16,), jnp.float32)

    @pl.loop(0, 128 // 16)
    def _ofill(i):
        ones[pl.ds(i * 16, 16)] = jnp.ones((16,), jnp.float32)

    zcopies = [
        pltpu.make_async_copy(
            zbuf, out_hbm.at[pl.ds(base + k * ZWORDS, ZWORDS)], semz
        )
        for k in range(NZD)
    ]
    for cp in zcopies:
        cp.start()

    pltpu.sync_copy(x_hbm.at[pl.ds(wid * XW, XW)], xv)

    lane = lax.iota(jnp.int32, 16)

    @pl.loop(0, XW // 16)
    def _offsets(t):
        v = xv[pl.ds(t * 16, 16)]
        b = wid * BPW + t // 64
        p = (t % 64) * 16
        off = v * PIX + (b * (KCLS * PIX) + p + lane)
        idxv[t // 8, pl.ds((t % 8) * 16, 16)] = off

    for cp in zcopies:
        cp.wait()

    scopies = [
        pltpu.make_async_copy(ones, out_hbm.at[idxv.at[j]], sems)
        for j in range(NSCAT)
    ]
    for cp in scopies:
        cp.start()
    for cp in scopies:
        cp.wait()


@functools.partial(jax.jit, static_argnums=())
def _run(x1d):
    fn = pl.kernel(
        _sc_body,
        out_type=jax.ShapeDtypeStruct((OUT_WORDS,), jnp.float32),
        mesh=plsc.VectorSubcoreMesh(core_axis_name="c", subcore_axis_name="s"),
        scratch_types=[
            pltpu.VMEM((XW,), jnp.int32),
            pltpu.VMEM((NSCAT, 128), jnp.int32),
            pltpu.VMEM((ZWORDS,), jnp.float32),
            pltpu.VMEM((128,), jnp.float32),
            pltpu.SemaphoreType.DMA,
            pltpu.SemaphoreType.DMA,
        ],
    )
    return fn(x1d)


def kernel(x):
    x1d = x.astype(jnp.int32).reshape(BATCH * PIX)
    out = _run(x1d)
    return out.reshape(BATCH, KCLS, 32, 32)


# speedup vs baseline: 1.0299x; 1.0299x over previous
import functools

import jax
import jax.numpy as jnp
from jax import lax
from jax.experimental import pallas as pl
from jax.experimental.pallas import tpu as pltpu
from jax.experimental.pallas import tpu_sc as plsc

KCLS = 128
BATCH = 64
PIX = 32 * 32
OUT_WORDS = BATCH * KCLS * PIX

NC = 2
NS = 16
NW = NC * NS
BPW = BATCH // NW
XW = BPW * PIX
NSCAT = XW // 128


def _sc_body(out_ref, x_hbm, xv, idxv, ones, sems):
    cid = lax.axis_index("c")
    sid = lax.axis_index("s")
    wid = sid * NC + cid

    @pl.loop(0, 128 // 16)
    def _ofill(i):
        ones[pl.ds(i * 16, 16)] = jnp.ones((16,), jnp.float32)

    pltpu.sync_copy(x_hbm.at[pl.ds(wid * XW, XW)], xv)

    lane = lax.iota(jnp.int32, 16)

    @pl.loop(0, XW // 16)
    def _offsets(t):
        v = xv[pl.ds(t * 16, 16)]
        b = wid * BPW + t // 64
        p = (t % 64) * 16
        off = v * PIX + (b * (KCLS * PIX) + p + lane)
        idxv[t // 8, pl.ds((t % 8) * 16, 16)] = off

    scopies = [
        pltpu.make_async_copy(ones, out_ref.at[idxv.at[j]], sems)
        for j in range(NSCAT)
    ]
    for cp in scopies:
        cp.start()
    for cp in scopies:
        cp.wait()


_scatter = pl.kernel(
    _sc_body,
    out_type=(),
    mesh=plsc.VectorSubcoreMesh(core_axis_name="c", subcore_axis_name="s"),
    scratch_types=[
        pltpu.VMEM((XW,), jnp.int32),
        pltpu.VMEM((NSCAT, 128), jnp.int32),
        pltpu.VMEM((128,), jnp.float32),
        pltpu.SemaphoreType.DMA,
    ],
)


def kernel(x):
    x1d = x.astype(jnp.int32).reshape(BATCH * PIX)
    out_ref = jax.new_ref(jnp.zeros((OUT_WORDS,), jnp.float32))
    _scatter(out_ref, x1d)
    return out_ref[...].reshape(BATCH, KCLS, 32, 32)


# speedup vs baseline: 4.6380x; 4.5032x over previous
import jax
import jax.numpy as jnp
from jax import lax
from jax.experimental import pallas as pl
from jax.experimental.pallas import tpu as pltpu

KCLS = 128
BATCH = 64
PIX = 32 * 32


def _onehot_body(x_ref, o_ref):
    x = x_ref[0]
    cls = lax.broadcasted_iota(jnp.int32, o_ref.shape, 1)
    o_ref[...] = (x[:, None, :] == cls).astype(jnp.float32)


def kernel(x):
    bb = 4
    x = x.astype(jnp.int32).reshape(BATCH // bb, bb, PIX)
    out = pl.pallas_call(
        _onehot_body,
        grid=(BATCH // bb,),
        in_specs=[pl.BlockSpec((1, bb, PIX), lambda i: (i, 0, 0))],
        out_specs=pl.BlockSpec((bb, KCLS, PIX), lambda i: (i, 0, 0)),
        out_shape=jax.ShapeDtypeStruct((BATCH, KCLS, PIX), jnp.float32),
        compiler_params=pltpu.CompilerParams(
            dimension_semantics=("arbitrary",),
        ),
    )(x)
    return out.reshape(BATCH, KCLS, 32, 32)


# speedup vs baseline: 4.9495x; 1.0671x over previous
import jax
import jax.numpy as jnp
from jax import lax
from jax.experimental import pallas as pl
from jax.experimental.pallas import tpu as pltpu

KCLS = 128
BATCH = 64
PIX = 32 * 32


def _onehot_body(x_ref, o_ref):
    x = x_ref[0]
    cls = lax.broadcasted_iota(jnp.int32, o_ref.shape, 1)
    o_ref[...] = (x[:, None, :] == cls).astype(jnp.float32)


def kernel(x):
    bb = 8
    x = x.astype(jnp.int32).reshape(BATCH // bb, bb, PIX)
    out = pl.pallas_call(
        _onehot_body,
        grid=(BATCH // bb,),
        in_specs=[pl.BlockSpec((1, bb, PIX), lambda i: (i, 0, 0))],
        out_specs=pl.BlockSpec((bb, KCLS, PIX), lambda i: (i, 0, 0)),
        out_shape=jax.ShapeDtypeStruct((BATCH, KCLS, PIX), jnp.float32),
        compiler_params=pltpu.CompilerParams(
            dimension_semantics=("arbitrary",),
        ),
    )(x)
    return out.reshape(BATCH, KCLS, 32, 32)
